# Initial kernel scaffold; baseline (speedup 1.0000x reference)
import functools

import jax
import jax.numpy as jnp
from jax import lax
from jax.experimental import pallas as pl
from jax.experimental.pallas import tpu as pltpu
from jax.experimental.pallas import tpu_sc as plsc

VOCAB = 1024
EMBED = 64
N_ROWS = 16 * 32 * 32
ROWS_PER_BLOCK = 2048
NUM_BLOCKS = N_ROWS // ROWS_PER_BLOCK

NUM_SC_CORES = 2
NUM_SUBCORES = 16
NUM_WORKERS = NUM_SC_CORES * NUM_SUBCORES
ROWS_PER_WORKER = N_ROWS // NUM_WORKERS


def _argmin_body(z_ref, cb_ref, idx_ref):
    z = z_ref[...]
    cb = cb_ref[...]
    z_sq = jnp.sum(z * z, axis=1, keepdims=True)
    e_sq = jnp.sum(cb * cb, axis=1, keepdims=True)
    scores = lax.dot_general(z, cb, (((1,), (1,)), ((), ())))
    d = z_sq + e_sq.T - 2.0 * scores
    m = jnp.min(d, axis=1, keepdims=True)
    j = lax.broadcasted_iota(jnp.int32, d.shape, 1)
    first = jnp.min(jnp.where(d == m, j, VOCAB), axis=1)
    idx_ref[0, 0, :] = first


def _argmin_indices(z_flat, codebook):
    out = pl.pallas_call(
        _argmin_body,
        grid=(NUM_BLOCKS,),
        in_specs=[
            pl.BlockSpec((ROWS_PER_BLOCK, EMBED), lambda i: (i, 0)),
            pl.BlockSpec((VOCAB, EMBED), lambda i: (0, 0)),
        ],
        out_specs=pl.BlockSpec((1, 1, ROWS_PER_BLOCK), lambda i: (i, 0, 0)),
        out_shape=jax.ShapeDtypeStruct((NUM_BLOCKS, 1, ROWS_PER_BLOCK),
                                       jnp.int32),
    )(z_flat, codebook)
    return out.reshape(-1)


_SC_MESH = plsc.VectorSubcoreMesh(core_axis_name="c", subcore_axis_name="s")


@functools.partial(
    pl.kernel,
    mesh=_SC_MESH,
    out_type=jax.ShapeDtypeStruct((N_ROWS, EMBED), jnp.float32),
    scratch_types=[
        pltpu.VMEM((ROWS_PER_WORKER,), jnp.int32),
        pltpu.VMEM((ROWS_PER_WORKER, EMBED), jnp.float32),
        pltpu.SemaphoreType.DMA,
    ],
)
def _sc_gather(cb_hbm, idx_hbm, out_hbm, idx_v, rows_v, sem):
    wid = lax.axis_index("s") * NUM_SC_CORES + lax.axis_index("c")
    base = wid * ROWS_PER_WORKER
    pltpu.sync_copy(idx_hbm.at[pl.ds(base, ROWS_PER_WORKER)], idx_v)
    pltpu.async_copy(cb_hbm.at[idx_v], rows_v, sem).wait()
    pltpu.sync_copy(rows_v, out_hbm.at[pl.ds(base, ROWS_PER_WORKER)])


def kernel(z, codebook):
    B, H, W, D = z.shape
    z_flat = z.reshape(-1, D)
    flat_idx = _argmin_indices(z_flat, codebook)
    indices = flat_idx.reshape(B, H, W)
    z_q = _sc_gather(codebook, flat_idx).reshape(B, H, W, D)
    return (z_q, indices)


# speedup vs baseline: 1.4740x; 1.4740x over previous
import functools

import jax
import jax.numpy as jnp
from jax import lax
from jax.experimental import pallas as pl
from jax.experimental.pallas import tpu as pltpu
from jax.experimental.pallas import tpu_sc as plsc

VOCAB = 1024
EMBED = 64
N_ROWS = 16 * 32 * 32
ROWS_PER_BLOCK = 2048
NUM_BLOCKS = N_ROWS // ROWS_PER_BLOCK

NUM_SC_CORES = 2
NUM_SUBCORES = 16
NUM_WORKERS = NUM_SC_CORES * NUM_SUBCORES
ROWS_PER_WORKER = N_ROWS // NUM_WORKERS


def _argmin_body(z_ref, cb_ref, idx_ref):
    z = z_ref[...]
    cb = cb_ref[...]
    z_sq = jnp.sum(z * z, axis=1, keepdims=True)
    e_sq = jnp.sum(cb * cb, axis=1, keepdims=True)
    scores = lax.dot_general(z, cb, (((1,), (1,)), ((), ())))
    d = z_sq + e_sq.T - 2.0 * scores
    m = jnp.min(d, axis=1, keepdims=True)
    j = lax.broadcasted_iota(jnp.int32, d.shape, 1)
    first = jnp.min(jnp.where(d == m, j, VOCAB), axis=1)
    idx_ref[0, 0, :] = first


def _argmin_indices(z_flat, codebook):
    out = pl.pallas_call(
        _argmin_body,
        grid=(NUM_BLOCKS,),
        in_specs=[
            pl.BlockSpec((ROWS_PER_BLOCK, EMBED), lambda i: (i, 0)),
            pl.BlockSpec((VOCAB, EMBED), lambda i: (0, 0)),
        ],
        out_specs=pl.BlockSpec((1, 1, ROWS_PER_BLOCK), lambda i: (i, 0, 0)),
        out_shape=jax.ShapeDtypeStruct((NUM_BLOCKS, 1, ROWS_PER_BLOCK),
                                       jnp.int32),
    )(z_flat, codebook)
    return out.reshape(-1)


@functools.cache
def _make_sc_gather():
    mesh = plsc.VectorSubcoreMesh(core_axis_name="c", subcore_axis_name="s")

    @functools.partial(
        pl.kernel,
        mesh=mesh,
        out_type=jax.ShapeDtypeStruct((N_ROWS, EMBED), jnp.float32),
        scratch_types=[
            pltpu.VMEM((ROWS_PER_WORKER,), jnp.int32),
            pltpu.VMEM((ROWS_PER_WORKER, EMBED), jnp.float32),
            pltpu.SemaphoreType.DMA,
        ],
        compiler_params=pltpu.CompilerParams(use_tc_tiling_on_sc=False),
    )
    def _sc_gather(cb_hbm, idx_hbm, out_hbm, idx_v, rows_v, sem):
        wid = lax.axis_index("s") * NUM_SC_CORES + lax.axis_index("c")
        base = wid * ROWS_PER_WORKER
        pltpu.sync_copy(idx_hbm.at[pl.ds(base, ROWS_PER_WORKER)], idx_v)
        pltpu.async_copy(cb_hbm.at[idx_v], rows_v, sem).wait()
        pltpu.sync_copy(rows_v, out_hbm.at[pl.ds(base, ROWS_PER_WORKER)])

    return _sc_gather


def kernel(z, codebook):
    B, H, W, D = z.shape
    z_flat = z.reshape(-1, D)
    flat_idx = _argmin_indices(z_flat, codebook)
    indices = flat_idx.reshape(B, H, W)
    z_q = _make_sc_gather()(codebook, flat_idx).reshape(B, H, W, D)
    return (z_q, indices)


# speedup vs baseline: 1.4761x; 1.0014x over previous
import functools

import jax
import jax.numpy as jnp
from jax import lax
from jax.experimental import pallas as pl
from jax.experimental.pallas import tpu as pltpu
from jax.experimental.pallas import tpu_sc as plsc

VOCAB = 1024
EMBED = 64
N_ROWS = 16 * 32 * 32
ROWS_PER_BLOCK = 2048
NUM_BLOCKS = N_ROWS // ROWS_PER_BLOCK

NUM_SC_CORES = 2
NUM_SUBCORES = 16
NUM_WORKERS = NUM_SC_CORES * NUM_SUBCORES
ROWS_PER_WORKER = N_ROWS // NUM_WORKERS


def _argmin_body(z_ref, cb_ref, idx_ref):
    z = z_ref[...]
    cb = cb_ref[...]
    z_sq = jnp.sum(z * z, axis=1, keepdims=True)
    e_sq = jnp.sum(cb * cb, axis=1, keepdims=True)
    scores = lax.dot_general(z, cb, (((1,), (1,)), ((), ())))
    d = z_sq + e_sq.T - 2.0 * scores
    m = jnp.min(d, axis=1, keepdims=True)
    j = lax.broadcasted_iota(jnp.int32, (1, VOCAB), 1).astype(jnp.float32)
    first = jnp.min(jnp.where(d == m, j, float(VOCAB)), axis=1)
    idx_ref[0, 0, :] = first.astype(jnp.int32)


def _argmin_indices(z_flat, codebook):
    out = pl.pallas_call(
        _argmin_body,
        grid=(NUM_BLOCKS,),
        in_specs=[
            pl.BlockSpec((ROWS_PER_BLOCK, EMBED), lambda i: (i, 0)),
            pl.BlockSpec((VOCAB, EMBED), lambda i: (0, 0)),
        ],
        out_specs=pl.BlockSpec((1, 1, ROWS_PER_BLOCK), lambda i: (i, 0, 0)),
        out_shape=jax.ShapeDtypeStruct((NUM_BLOCKS, 1, ROWS_PER_BLOCK),
                                       jnp.int32),
    )(z_flat, codebook)
    return out.reshape(-1)


@functools.cache
def _make_sc_gather():
    mesh = plsc.VectorSubcoreMesh(core_axis_name="c", subcore_axis_name="s")

    @functools.partial(
        pl.kernel,
        mesh=mesh,
        out_type=jax.ShapeDtypeStruct((N_ROWS, EMBED), jnp.float32),
        scratch_types=[
            pltpu.VMEM((ROWS_PER_WORKER,), jnp.int32),
            pltpu.VMEM((ROWS_PER_WORKER, EMBED), jnp.float32),
            pltpu.SemaphoreType.DMA,
        ],
        compiler_params=pltpu.CompilerParams(use_tc_tiling_on_sc=False),
    )
    def _sc_gather(cb_hbm, idx_hbm, out_hbm, idx_v, rows_v, sem):
        wid = lax.axis_index("s") * NUM_SC_CORES + lax.axis_index("c")
        base = wid * ROWS_PER_WORKER
        pltpu.sync_copy(idx_hbm.at[pl.ds(base, ROWS_PER_WORKER)], idx_v)
        pltpu.async_copy(cb_hbm.at[idx_v], rows_v, sem).wait()
        pltpu.sync_copy(rows_v, out_hbm.at[pl.ds(base, ROWS_PER_WORKER)])

    return _sc_gather


def kernel(z, codebook):
    B, H, W, D = z.shape
    z_flat = z.reshape(-1, D)
    flat_idx = _argmin_indices(z_flat, codebook)
    indices = flat_idx.reshape(B, H, W)
    z_q = _make_sc_gather()(codebook, flat_idx).reshape(B, H, W, D)
    return (z_q, indices)


# speedup vs baseline: 1.8786x; 1.2727x over previous
import functools

import jax
import jax.numpy as jnp
from jax import lax
from jax.experimental import pallas as pl
from jax.experimental.pallas import tpu as pltpu
from jax.experimental.pallas import tpu_sc as plsc

VOCAB = 1024
EMBED = 64
N_ROWS = 16 * 32 * 32
ROWS_PER_BLOCK = 2048
NUM_BLOCKS = N_ROWS // ROWS_PER_BLOCK

NUM_SC_CORES = 2
NUM_SUBCORES = 16
NUM_WORKERS = NUM_SC_CORES * NUM_SUBCORES
ROWS_PER_WORKER = N_ROWS // NUM_WORKERS


def _argmin_body(z_ref, cb_ref, idx_ref):
    z = z_ref[...]
    cb = cb_ref[...]
    z_sq = jnp.sum(z * z, axis=1, keepdims=True)
    e_sq = jnp.sum(cb * cb, axis=1, keepdims=True)
    scores2 = lax.dot_general(cb, z + z, (((1,), (1,)), ((), ())))
    d = e_sq + z_sq.T - scores2
    first = jnp.argmin(d, axis=0)
    idx_ref[0, 0, :] = first.astype(jnp.int32)


def _argmin_indices(z_flat, codebook):
    out = pl.pallas_call(
        _argmin_body,
        grid=(NUM_BLOCKS,),
        in_specs=[
            pl.BlockSpec((ROWS_PER_BLOCK, EMBED), lambda i: (i, 0)),
            pl.BlockSpec((VOCAB, EMBED), lambda i: (0, 0)),
        ],
        out_specs=pl.BlockSpec((1, 1, ROWS_PER_BLOCK), lambda i: (i, 0, 0)),
        out_shape=jax.ShapeDtypeStruct((NUM_BLOCKS, 1, ROWS_PER_BLOCK),
                                       jnp.int32),
    )(z_flat, codebook)
    return out.reshape(-1)


@functools.cache
def _make_sc_gather():
    mesh = plsc.VectorSubcoreMesh(core_axis_name="c", subcore_axis_name="s")

    @functools.partial(
        pl.kernel,
        mesh=mesh,
        out_type=jax.ShapeDtypeStruct((N_ROWS, EMBED), jnp.float32),
        scratch_types=[
            pltpu.VMEM((ROWS_PER_WORKER,), jnp.int32),
            pltpu.VMEM((ROWS_PER_WORKER, EMBED), jnp.float32),
            pltpu.SemaphoreType.DMA,
        ],
        compiler_params=pltpu.CompilerParams(use_tc_tiling_on_sc=False),
    )
    def _sc_gather(cb_hbm, idx_hbm, out_hbm, idx_v, rows_v, sem):
        wid = lax.axis_index("s") * NUM_SC_CORES + lax.axis_index("c")
        base = wid * ROWS_PER_WORKER
        pltpu.sync_copy(idx_hbm.at[pl.ds(base, ROWS_PER_WORKER)], idx_v)
        pltpu.async_copy(cb_hbm.at[idx_v], rows_v, sem).wait()
        pltpu.sync_copy(rows_v, out_hbm.at[pl.ds(base, ROWS_PER_WORKER)])

    return _sc_gather


def kernel(z, codebook):
    B, H, W, D = z.shape
    z_flat = z.reshape(-1, D)
    flat_idx = _argmin_indices(z_flat, codebook)
    indices = flat_idx.reshape(B, H, W)
    z_q = _make_sc_gather()(codebook, flat_idx).reshape(B, H, W, D)
    return (z_q, indices)


# speedup vs baseline: 2.0844x; 1.1095x over previous
import functools

import jax
import jax.numpy as jnp
from jax import lax
from jax.experimental import pallas as pl
from jax.experimental.pallas import tpu as pltpu
from jax.experimental.pallas import tpu_sc as plsc

VOCAB = 1024
EMBED = 64
N_ROWS = 16 * 32 * 32
ROWS_PER_BLOCK = 2048
NUM_BLOCKS = N_ROWS // ROWS_PER_BLOCK

NUM_SC_CORES = 2
NUM_SUBCORES = 16
NUM_WORKERS = NUM_SC_CORES * NUM_SUBCORES
ROWS_PER_WORKER = N_ROWS // NUM_WORKERS
IMGS_PER_BLOCK = ROWS_PER_BLOCK // VOCAB


def _argmin_body(z_ref, cb_ref, idx_ref, cbp_ref):
    z = z_ref[...]
    cb = cb_ref[...]
    z_sq = jnp.sum(z * z, axis=1, keepdims=True)
    e_sq = jnp.sum(cb * cb, axis=1, keepdims=True)
    scores2 = lax.dot_general(cb, z + z, (((1,), (1,)), ((), ())))
    d = e_sq + z_sq.T - scores2
    first = jnp.argmin(d, axis=0)
    idx_ref[0, :, :] = first.astype(jnp.int32).reshape(IMGS_PER_BLOCK, VOCAB)

    @pl.when(pl.program_id(0) == 0)
    def _():
        cbp_ref[...] = jnp.concatenate(
            [cb, jnp.zeros((VOCAB, 128 - EMBED), jnp.float32)], axis=1)


def _argmin_indices(z_flat, codebook):
    idx, cb_pad = pl.pallas_call(
        _argmin_body,
        grid=(NUM_BLOCKS,),
        in_specs=[
            pl.BlockSpec((ROWS_PER_BLOCK, EMBED), lambda i: (i, 0)),
            pl.BlockSpec((VOCAB, EMBED), lambda i: (0, 0)),
        ],
        out_specs=[
            pl.BlockSpec((1, IMGS_PER_BLOCK, VOCAB), lambda i: (i, 0, 0)),
            pl.BlockSpec((VOCAB, 128), lambda i: (0, 0)),
        ],
        out_shape=[
            jax.ShapeDtypeStruct((NUM_BLOCKS, IMGS_PER_BLOCK, VOCAB),
                                 jnp.int32),
            jax.ShapeDtypeStruct((VOCAB, 128), jnp.float32),
        ],
    )(z_flat, codebook)
    return idx, cb_pad


@functools.cache
def _make_sc_gather():
    mesh = plsc.VectorSubcoreMesh(core_axis_name="c", subcore_axis_name="s")

    @functools.partial(
        pl.kernel,
        mesh=mesh,
        out_type=jax.ShapeDtypeStruct((N_ROWS, 128), jnp.float32),
        scratch_types=[
            pltpu.VMEM((ROWS_PER_WORKER,), jnp.int32),
            pltpu.VMEM((ROWS_PER_WORKER, 128), jnp.float32),
            pltpu.SemaphoreType.DMA,
        ],
    )
    def _sc_gather(cbp_hbm, idx_hbm, out_hbm, idx_v, rows_v, sem):
        wid = lax.axis_index("s") * NUM_SC_CORES + lax.axis_index("c")
        base = wid * ROWS_PER_WORKER
        blk = base // ROWS_PER_BLOCK
        rem = base % ROWS_PER_BLOCK
        img = rem // VOCAB
        off = rem % VOCAB
        pltpu.sync_copy(idx_hbm.at[blk, img, pl.ds(off, ROWS_PER_WORKER)],
                        idx_v)
        pltpu.async_copy(cbp_hbm.at[idx_v], rows_v, sem).wait()
        pltpu.sync_copy(rows_v, out_hbm.at[pl.ds(base, ROWS_PER_WORKER)])

    return _sc_gather


def kernel(z, codebook):
    B, H, W, D = z.shape
    z_flat = z.reshape(-1, D)
    idx, cb_pad = _argmin_indices(z_flat, codebook)
    indices = idx.reshape(B, H, W)
    z_q = _make_sc_gather()(cb_pad, idx)[:, :D].reshape(B, H, W, D)
    return (z_q, indices)
